# Initial kernel scaffold; baseline (speedup 1.0000x reference)
import jax
import jax.numpy as jnp
from jax.experimental import pallas as pl


def kernel(x, gate_W, gate_b):
    raise NotImplementedError("write your pallas kernel here")


# speedup vs baseline: 3.2758x; 3.2758x over previous
import jax
import jax.numpy as jnp
from jax.experimental import pallas as pl
from jax.experimental.pallas import tpu as pltpu

BM = 512


def _router_kernel(x_ref, w_ref, b_ref, gating_ref, idx_ref):
    logits = (
        jnp.dot(x_ref[:].astype(jnp.bfloat16), w_ref[:].astype(jnp.bfloat16),
                preferred_element_type=jnp.float32)
        + b_ref[:]
    )
    row_max = jnp.max(logits, axis=-1, keepdims=True)
    e = jnp.exp(logits - row_max)
    denom = jnp.sum(e, axis=-1, keepdims=True)
    gating_ref[:] = e / denom
    iota = jax.lax.broadcasted_iota(jnp.int32, logits.shape, 1)
    cand = jnp.where(logits == row_max, iota, jnp.int32(2**30))
    idx_ref[:] = jnp.min(cand, axis=-1, keepdims=True)


def kernel(x, gate_W, gate_b):
    B, S, D = x.shape
    M = B * S
    x2 = x.reshape(M, D)
    b2 = gate_b.reshape(1, D)
    grid = (M // BM,)
    gating, idx = pl.pallas_call(
        _router_kernel,
        grid=grid,
        in_specs=[
            pl.BlockSpec((BM, D), lambda i: (i, 0)),
            pl.BlockSpec((D, D), lambda i: (0, 0)),
            pl.BlockSpec((1, D), lambda i: (0, 0)),
        ],
        out_specs=[
            pl.BlockSpec((BM, D), lambda i: (i, 0)),
            pl.BlockSpec((BM, 1), lambda i: (i, 0)),
        ],
        out_shape=[
            jax.ShapeDtypeStruct((M, D), jnp.float32),
            jax.ShapeDtypeStruct((M, 1), jnp.int32),
        ],
        compiler_params=pltpu.CompilerParams(
            dimension_semantics=("arbitrary",),
        ),
    )(x2, gate_W, b2)
    return gating.reshape(B, S, D), idx.reshape(B, S)
